# Initial kernel scaffold; baseline (speedup 1.0000x reference)
import jax
import jax.numpy as jnp
from jax.experimental import pallas as pl


def kernel(features, edge_index, W1, b1, W2, b2):
    raise NotImplementedError("write your pallas kernel here")


# speedup vs baseline: 6.2604x; 6.2604x over previous
import functools

import jax
import jax.numpy as jnp
from jax import lax
from jax.experimental import pallas as pl
from jax.experimental.pallas import tpu as pltpu
from jax.experimental.pallas import tpu_sc as plsc

NS = 16
NCORES = 2


def _deg_kernel(NP, E, C):
    RPT = E // C // NS
    NPT = NP // NS
    mesh = plsc.VectorSubcoreMesh(core_axis_name="c", subcore_axis_name="s")

    @functools.partial(
        pl.kernel,
        out_type=[
            jax.ShapeDtypeStruct((NP,), jnp.float32),
            jax.ShapeDtypeStruct((NP,), jnp.float32),
        ],
        mesh=mesh,
        compiler_params=pltpu.CompilerParams(use_tc_tiling_on_sc=False),
        scratch_types=[
            pltpu.VMEM((RPT, C), jnp.int32),
            pltpu.VMEM((C,), jnp.float32),
            pltpu.VMEM((NPT,), jnp.float32),
            pltpu.VMEM_SHARED((NP,), jnp.float32),
        ],
    )
    def deg(ei4, ones_h, zeros_h, out0_h, out1_h, idx_v, ones_v, zer_v, acc_s):
        c = lax.axis_index("c")
        s = lax.axis_index("s")
        pltpu.sync_copy(zeros_h, zer_v)
        pltpu.sync_copy(zer_v, acc_s.at[pl.ds(s * NPT, NPT)])
        pltpu.sync_copy(ones_h, ones_v)
        pltpu.sync_copy(ei4.at[c, s], idx_v)
        plsc.subcore_barrier()

        @pl.loop(0, RPT)
        def _(j):
            pltpu.sync_copy(ones_v, acc_s.at[idx_v.at[j]], add=True)

        plsc.subcore_barrier()
        sl = pl.ds(s * NPT, NPT)
        pltpu.sync_copy(acc_s.at[sl], zer_v)

        @pl.when(c == 0)
        def _():
            pltpu.sync_copy(zer_v, out0_h.at[sl])

        @pl.when(c == 1)
        def _():
            pltpu.sync_copy(zer_v, out1_h.at[sl])

    return deg


def _agg_dsplit_kernel(NP, DHALF, E, C):
    RPT = E // C // NS
    NPT = NP // NS
    mesh = plsc.VectorSubcoreMesh(core_axis_name="c", subcore_axis_name="s")

    @functools.partial(
        pl.kernel,
        out_type=jax.ShapeDtypeStruct((NCORES, NP, DHALF), jnp.float32),
        mesh=mesh,
        compiler_params=pltpu.CompilerParams(use_tc_tiling_on_sc=False),
        scratch_types=[
            pltpu.VMEM((RPT, C), jnp.int32),
            pltpu.VMEM((RPT, C), jnp.int32),
            pltpu.VMEM((C, DHALF), jnp.float32),
            pltpu.VMEM((NPT, DHALF), jnp.float32),
            pltpu.VMEM_SHARED((NP, DHALF), jnp.float32),
            pltpu.SemaphoreType.DMA,
        ],
    )
    def agg(ha_h, hb_h, src3, dst3, zeros_h, out_h,
            src_v, dst_v, rows_v, buf_v, acc_s, sem):
        c = lax.axis_index("c")
        s = lax.axis_index("s")
        pltpu.sync_copy(zeros_h, buf_v)
        pltpu.sync_copy(buf_v, acc_s.at[pl.ds(s * NPT, NPT)])
        pltpu.sync_copy(src3.at[s], src_v)
        pltpu.sync_copy(dst3.at[s], dst_v)
        plsc.subcore_barrier()

        def edge_loop(h_h):
            @pl.loop(0, RPT)
            def _(j):
                pltpu.async_copy(h_h.at[src_v.at[j]], rows_v, sem).wait()
                pltpu.sync_copy(rows_v, acc_s.at[dst_v.at[j]], add=True)

        @pl.when(c == 0)
        def _():
            edge_loop(ha_h)

        @pl.when(c == 1)
        def _():
            edge_loop(hb_h)

        plsc.subcore_barrier()
        pltpu.sync_copy(acc_s.at[pl.ds(s * NPT, NPT)], buf_v)
        pltpu.sync_copy(buf_v, out_h.at[c, pl.ds(s * NPT, NPT)])

    return agg


def _agg_esplit_kernel(NP, D, E, C):
    RPT = E // NCORES // C // NS
    NPT = NP // NS
    mesh = plsc.VectorSubcoreMesh(core_axis_name="c", subcore_axis_name="s")

    @functools.partial(
        pl.kernel,
        out_type=jax.ShapeDtypeStruct((NCORES, NP, D), jnp.float32),
        mesh=mesh,
        compiler_params=pltpu.CompilerParams(use_tc_tiling_on_sc=False),
        scratch_types=[
            pltpu.VMEM((RPT, C), jnp.int32),
            pltpu.VMEM((RPT, C), jnp.int32),
            pltpu.VMEM((C, D), jnp.float32),
            pltpu.VMEM((NPT, D), jnp.float32),
            pltpu.VMEM_SHARED((NP, D), jnp.float32),
            pltpu.SemaphoreType.DMA,
        ],
    )
    def agg(h_h, src3, dst3, zeros_h, out_h, src_v, dst_v, rows_v, buf_v, acc_s, sem):
        c = lax.axis_index("c")
        s = lax.axis_index("s")
        w = c * NS + s
        pltpu.sync_copy(zeros_h, buf_v)
        pltpu.sync_copy(buf_v, acc_s.at[pl.ds(s * NPT, NPT)])
        pltpu.sync_copy(src3.at[w], src_v)
        pltpu.sync_copy(dst3.at[w], dst_v)
        plsc.subcore_barrier()

        @pl.loop(0, RPT)
        def _(j):
            pltpu.async_copy(h_h.at[src_v.at[j]], rows_v, sem).wait()
            pltpu.sync_copy(rows_v, acc_s.at[dst_v.at[j]], add=True)

        plsc.subcore_barrier()
        pltpu.sync_copy(acc_s.at[pl.ds(s * NPT, NPT)], buf_v)
        pltpu.sync_copy(buf_v, out_h.at[c, pl.ds(s * NPT, NPT)])

    return agg


def _mm1(x, dego, W1h, B=512):
    N, DI = x.shape
    DH = W1h.shape[1]

    def body(x_ref, d_ref, w_ref, o_ref):
        nrm = lax.rsqrt(jnp.maximum(d_ref[...], 1.0))
        o_ref[...] = jnp.dot(
            x_ref[...] * nrm, w_ref[...], preferred_element_type=jnp.float32
        )

    return pl.pallas_call(
        body,
        grid=(pl.cdiv(N, B),),
        in_specs=[
            pl.BlockSpec((B, DI), lambda i: (i, 0)),
            pl.BlockSpec((B, 1), lambda i: (i, 0)),
            pl.BlockSpec((DI, DH), lambda i: (0, 0)),
        ],
        out_specs=pl.BlockSpec((B, DH), lambda i: (i, 0)),
        out_shape=jax.ShapeDtypeStruct((N, DH), jnp.float32),
    )(x, dego, W1h)


def _mm2(p, degi, dego, b1, W2, N, B=512):
    _, NP, DHALF = p.shape
    DH = 2 * DHALF
    K = W2.shape[1]

    def body(p_ref, di_ref, do_ref, b_ref, w_ref, o_ref):
        agg = jnp.concatenate([p_ref[0], p_ref[1]], axis=1)
        ndst = lax.rsqrt(jnp.maximum(di_ref[...], 1.0))
        h = jnp.maximum(agg * ndst + b_ref[...], 0.0)
        nsrc = lax.rsqrt(jnp.maximum(do_ref[...], 1.0))
        o_ref[...] = jnp.dot(h * nsrc, w_ref[...], preferred_element_type=jnp.float32)

    return pl.pallas_call(
        body,
        grid=(pl.cdiv(N, B),),
        in_specs=[
            pl.BlockSpec((2, B, DHALF), lambda i: (0, i, 0)),
            pl.BlockSpec((B, 1), lambda i: (i, 0)),
            pl.BlockSpec((B, 1), lambda i: (i, 0)),
            pl.BlockSpec((1, DH), lambda i: (0, 0)),
            pl.BlockSpec((DH, K), lambda i: (0, 0)),
        ],
        out_specs=pl.BlockSpec((B, K), lambda i: (i, 0)),
        out_shape=jax.ShapeDtypeStruct((N, K), jnp.float32),
    )(p, degi, dego, b1, W2)


def _fin(p, degi, b2, N, B=512):
    _, NP, K = p.shape

    def body(p_ref, di_ref, b_ref, o_ref):
        agg = p_ref[0] + p_ref[1]
        ndst = lax.rsqrt(jnp.maximum(di_ref[...], 1.0))
        o_ref[...] = agg * ndst + b_ref[...]

    return pl.pallas_call(
        body,
        grid=(pl.cdiv(N, B),),
        in_specs=[
            pl.BlockSpec((2, B, K), lambda i: (0, i, 0)),
            pl.BlockSpec((B, 1), lambda i: (i, 0)),
            pl.BlockSpec((1, K), lambda i: (0, 0)),
        ],
        out_specs=pl.BlockSpec((B, K), lambda i: (i, 0)),
        out_shape=jax.ShapeDtypeStruct((N, K), jnp.float32),
    )(p, degi, b2)


def kernel(features, edge_index, W1, b1, W2, b2):
    N, DI = features.shape
    E = edge_index.shape[1]
    DH = W1.shape[1]
    K = W2.shape[1]
    NP = ((N + 127) // 128) * 128
    C = 80
    assert E % (NCORES * NS * C) == 0 and NP % (NS * 8) == 0 and DH % 2 == 0

    ei4 = edge_index.reshape(2, NS, E // NS // C, C)
    src3t = edge_index[0].reshape(NS, E // NS // C, C)
    dst3t = edge_index[1].reshape(NS, E // NS // C, C)
    src3w = edge_index[0].reshape(NCORES * NS, E // NCORES // NS // C, C)
    dst3w = edge_index[1].reshape(NCORES * NS, E // NCORES // NS // C, C)

    dego, degi = _deg_kernel(NP, E, C)(
        ei4, jnp.ones((C,), jnp.float32), jnp.zeros((NP // NS,), jnp.float32)
    )
    dego = dego.reshape(NP, 1)
    degi = degi.reshape(NP, 1)

    DHALF = DH // 2
    h1a = _mm1(features, dego, W1[:, :DHALF])
    h1b = _mm1(features, dego, W1[:, DHALF:])
    p1 = _agg_dsplit_kernel(NP, DHALF, E, C)(
        h1a, h1b, src3t, dst3t, jnp.zeros((NP // NS, DHALF), jnp.float32)
    )
    h2 = _mm2(p1, degi, dego, b1.reshape(1, DH), W2, N)
    p2 = _agg_esplit_kernel(NP, K, E, C)(
        h2, src3w, dst3w, jnp.zeros((NP // NS, K), jnp.float32)
    )
    return _fin(p2, degi, b2.reshape(1, K), N)


# speedup vs baseline: 9.3967x; 1.5010x over previous
import functools

import jax
import jax.numpy as jnp
from jax import lax
from jax.experimental import pallas as pl
from jax.experimental.pallas import tpu as pltpu
from jax.experimental.pallas import tpu_sc as plsc

NS = 16
NCORES = 2


def _deg_kernel(NP, E, C):
    RPT = E // C // NS
    NPT = NP // NS
    mesh = plsc.VectorSubcoreMesh(core_axis_name="c", subcore_axis_name="s")

    @functools.partial(
        pl.kernel,
        out_type=[
            jax.ShapeDtypeStruct((NP,), jnp.float32),
            jax.ShapeDtypeStruct((NP,), jnp.float32),
        ],
        mesh=mesh,
        compiler_params=pltpu.CompilerParams(use_tc_tiling_on_sc=False),
        scratch_types=[
            pltpu.VMEM((RPT, C), jnp.int32),
            pltpu.VMEM((C,), jnp.float32),
            pltpu.VMEM((NPT,), jnp.float32),
            pltpu.VMEM_SHARED((NP,), jnp.float32),
        ],
    )
    def deg(ei4, ones_h, zeros_h, out0_h, out1_h, idx_v, ones_v, zer_v, acc_s):
        c = lax.axis_index("c")
        s = lax.axis_index("s")
        pltpu.sync_copy(zeros_h, zer_v)
        pltpu.sync_copy(zer_v, acc_s.at[pl.ds(s * NPT, NPT)])
        pltpu.sync_copy(ones_h, ones_v)
        pltpu.sync_copy(ei4.at[c, s], idx_v)
        plsc.subcore_barrier()

        @pl.loop(0, RPT)
        def _(j):
            pltpu.sync_copy(ones_v, acc_s.at[idx_v.at[j]], add=True)

        plsc.subcore_barrier()
        sl = pl.ds(s * NPT, NPT)
        pltpu.sync_copy(acc_s.at[sl], zer_v)

        @pl.when(c == 0)
        def _():
            pltpu.sync_copy(zer_v, out0_h.at[sl])

        @pl.when(c == 1)
        def _():
            pltpu.sync_copy(zer_v, out1_h.at[sl])

    return deg


def _edge_pipeline(h_h, src_v, dst_v, rows_a, rows_b, sem_a, sem_b, acc_s, RPT):
    pairs = RPT - (RPT % 2)
    pltpu.async_copy(h_h.at[src_v.at[0]], rows_a, sem_a)

    @pl.loop(0, pairs, step=2)
    def _(j):
        pltpu.async_copy(h_h.at[src_v.at[j + 1]], rows_b, sem_b)
        pltpu.make_async_copy(h_h.at[src_v.at[j]], rows_a, sem_a).wait()
        pltpu.sync_copy(rows_a, acc_s.at[dst_v.at[j]], add=True)

        @pl.when(j + 2 < RPT)
        def _():
            pltpu.async_copy(h_h.at[src_v.at[j + 2]], rows_a, sem_a)

        pltpu.make_async_copy(h_h.at[src_v.at[j + 1]], rows_b, sem_b).wait()
        pltpu.sync_copy(rows_b, acc_s.at[dst_v.at[j + 1]], add=True)

    if RPT % 2:
        pltpu.make_async_copy(h_h.at[src_v.at[RPT - 1]], rows_a, sem_a).wait()
        pltpu.sync_copy(rows_a, acc_s.at[dst_v.at[RPT - 1]], add=True)


def _agg_dsplit_kernel(NP, DHALF, EP, C):
    RPT = EP // C // NS
    NPT = NP // NS
    ZCH = 128
    assert NPT % ZCH == 0
    mesh = plsc.VectorSubcoreMesh(core_axis_name="c", subcore_axis_name="s")

    @functools.partial(
        pl.kernel,
        out_type=jax.ShapeDtypeStruct((NCORES, NP, DHALF), jnp.float32),
        mesh=mesh,
        compiler_params=pltpu.CompilerParams(use_tc_tiling_on_sc=False),
        scratch_types=[
            pltpu.VMEM((RPT, C), jnp.int32),
            pltpu.VMEM((RPT, C), jnp.int32),
            pltpu.VMEM((C, DHALF), jnp.float32),
            pltpu.VMEM((C, DHALF), jnp.float32),
            pltpu.VMEM((ZCH, DHALF), jnp.float32),
            pltpu.VMEM_SHARED((NP, DHALF), jnp.float32),
            pltpu.SemaphoreType.DMA,
            pltpu.SemaphoreType.DMA,
        ],
    )
    def agg(ha_h, hb_h, src3, dst3, zeros_h, out_h,
            src_v, dst_v, rows_a, rows_b, buf_v, acc_s, sem_a, sem_b):
        c = lax.axis_index("c")
        s = lax.axis_index("s")
        pltpu.sync_copy(zeros_h, buf_v)

        @pl.loop(0, NPT // ZCH)
        def _(k):
            pltpu.sync_copy(buf_v, acc_s.at[pl.ds(s * NPT + k * ZCH, ZCH)])

        pltpu.sync_copy(src3.at[s], src_v)
        pltpu.sync_copy(dst3.at[s], dst_v)
        plsc.subcore_barrier()

        @pl.when(c == 0)
        def _():
            _edge_pipeline(ha_h, src_v, dst_v, rows_a, rows_b, sem_a, sem_b,
                           acc_s, RPT)

        @pl.when(c == 1)
        def _():
            _edge_pipeline(hb_h, src_v, dst_v, rows_a, rows_b, sem_a, sem_b,
                           acc_s, RPT)

        plsc.subcore_barrier()

        @pl.loop(0, NPT // ZCH)
        def _(k):
            pltpu.sync_copy(acc_s.at[pl.ds(s * NPT + k * ZCH, ZCH)], buf_v)
            pltpu.sync_copy(buf_v, out_h.at[c, pl.ds(s * NPT + k * ZCH, ZCH)])

    return agg


def _agg_esplit_kernel(NP, D, EP, C):
    RPT = EP // NCORES // C // NS
    NPT = NP // NS
    ZCH = 128
    assert NPT % ZCH == 0
    mesh = plsc.VectorSubcoreMesh(core_axis_name="c", subcore_axis_name="s")

    @functools.partial(
        pl.kernel,
        out_type=jax.ShapeDtypeStruct((NCORES, NP, D), jnp.float32),
        mesh=mesh,
        compiler_params=pltpu.CompilerParams(use_tc_tiling_on_sc=False),
        scratch_types=[
            pltpu.VMEM((RPT, C), jnp.int32),
            pltpu.VMEM((RPT, C), jnp.int32),
            pltpu.VMEM((C, D), jnp.float32),
            pltpu.VMEM((C, D), jnp.float32),
            pltpu.VMEM((ZCH, D), jnp.float32),
            pltpu.VMEM_SHARED((NP, D), jnp.float32),
            pltpu.SemaphoreType.DMA,
            pltpu.SemaphoreType.DMA,
        ],
    )
    def agg(h_h, src3, dst3, zeros_h, out_h,
            src_v, dst_v, rows_a, rows_b, buf_v, acc_s, sem_a, sem_b):
        c = lax.axis_index("c")
        s = lax.axis_index("s")
        w = c * NS + s
        pltpu.sync_copy(zeros_h, buf_v)

        @pl.loop(0, NPT // ZCH)
        def _(k):
            pltpu.sync_copy(buf_v, acc_s.at[pl.ds(s * NPT + k * ZCH, ZCH)])

        pltpu.sync_copy(src3.at[w], src_v)
        pltpu.sync_copy(dst3.at[w], dst_v)
        plsc.subcore_barrier()

        _edge_pipeline(h_h, src_v, dst_v, rows_a, rows_b, sem_a, sem_b, acc_s, RPT)

        plsc.subcore_barrier()

        @pl.loop(0, NPT // ZCH)
        def _(k):
            pltpu.sync_copy(acc_s.at[pl.ds(s * NPT + k * ZCH, ZCH)], buf_v)
            pltpu.sync_copy(buf_v, out_h.at[c, pl.ds(s * NPT + k * ZCH, ZCH)])

    return agg


def _mm1(x, dego, W1h, B=512):
    N, DI = x.shape
    DH = W1h.shape[1]

    def body(x_ref, d_ref, w_ref, o_ref):
        nrm = lax.rsqrt(jnp.maximum(d_ref[...], 1.0))
        o_ref[...] = jnp.dot(
            x_ref[...] * nrm, w_ref[...], preferred_element_type=jnp.float32
        )

    return pl.pallas_call(
        body,
        grid=(pl.cdiv(N, B),),
        in_specs=[
            pl.BlockSpec((B, DI), lambda i: (i, 0)),
            pl.BlockSpec((B, 1), lambda i: (i, 0)),
            pl.BlockSpec((DI, DH), lambda i: (0, 0)),
        ],
        out_specs=pl.BlockSpec((B, DH), lambda i: (i, 0)),
        out_shape=jax.ShapeDtypeStruct((N, DH), jnp.float32),
    )(x, dego, W1h)


def _mm2(p, degi, dego, b1, W2, N, B=512):
    _, NP, DHALF = p.shape
    DH = 2 * DHALF
    K = W2.shape[1]

    def body(p_ref, di_ref, do_ref, b_ref, w_ref, o_ref):
        agg = jnp.concatenate([p_ref[0], p_ref[1]], axis=1)
        ndst = lax.rsqrt(jnp.maximum(di_ref[...], 1.0))
        h = jnp.maximum(agg * ndst + b_ref[...], 0.0)
        nsrc = lax.rsqrt(jnp.maximum(do_ref[...], 1.0))
        o_ref[...] = jnp.dot(h * nsrc, w_ref[...], preferred_element_type=jnp.float32)

    return pl.pallas_call(
        body,
        grid=(pl.cdiv(N, B),),
        in_specs=[
            pl.BlockSpec((2, B, DHALF), lambda i: (0, i, 0)),
            pl.BlockSpec((B, 1), lambda i: (i, 0)),
            pl.BlockSpec((B, 1), lambda i: (i, 0)),
            pl.BlockSpec((1, DH), lambda i: (0, 0)),
            pl.BlockSpec((DH, K), lambda i: (0, 0)),
        ],
        out_specs=pl.BlockSpec((B, K), lambda i: (i, 0)),
        out_shape=jax.ShapeDtypeStruct((N, K), jnp.float32),
    )(p, degi, dego, b1, W2)


def _fin(p, degi, b2, N, B=512):
    _, NP, K = p.shape

    def body(p_ref, di_ref, b_ref, o_ref):
        agg = p_ref[0] + p_ref[1]
        ndst = lax.rsqrt(jnp.maximum(di_ref[...], 1.0))
        o_ref[...] = agg * ndst + b_ref[...]

    return pl.pallas_call(
        body,
        grid=(pl.cdiv(N, B),),
        in_specs=[
            pl.BlockSpec((2, B, K), lambda i: (0, i, 0)),
            pl.BlockSpec((B, 1), lambda i: (i, 0)),
            pl.BlockSpec((1, K), lambda i: (0, 0)),
        ],
        out_specs=pl.BlockSpec((B, K), lambda i: (i, 0)),
        out_shape=jax.ShapeDtypeStruct((N, K), jnp.float32),
    )(p, degi, b2)


def kernel(features, edge_index, W1, b1, W2, b2):
    N, DI = features.shape
    E = edge_index.shape[1]
    DH = W1.shape[1]
    K = W2.shape[1]
    NP = ((N + 2047) // 2048) * 2048
    CD = 80
    C1 = 80
    C2 = 80
    assert E % (NS * CD) == 0 and NP % (NS * 8) == 0
    assert DH % 2 == 0 and K % 2 == 0

    GR = NCORES * NS * C2
    EP = ((E + GR - 1) // GR) * GR
    srcp = jnp.concatenate([edge_index[0], jnp.zeros((EP - E,), jnp.int32)])
    dstp = jnp.concatenate([edge_index[1], jnp.full((EP - E,), NP - 1, jnp.int32)])

    ei4 = edge_index.reshape(2, NS, E // NS // CD, CD)
    src3t1 = srcp.reshape(NS, EP // NS // C1, C1)
    dst3t1 = dstp.reshape(NS, EP // NS // C1, C1)
    src3w = srcp.reshape(NCORES * NS, EP // NCORES // NS // C2, C2)
    dst3w = dstp.reshape(NCORES * NS, EP // NCORES // NS // C2, C2)

    dego, degi = _deg_kernel(NP, E, CD)(
        ei4, jnp.ones((CD,), jnp.float32), jnp.zeros((NP // NS,), jnp.float32)
    )
    dego = dego.reshape(NP, 1)
    degi = degi.reshape(NP, 1)

    DHALF = DH // 2
    h1a = _mm1(features, dego, W1[:, :DHALF])
    h1b = _mm1(features, dego, W1[:, DHALF:])
    p1 = _agg_dsplit_kernel(NP, DHALF, EP, C1)(
        h1a, h1b, src3t1, dst3t1, jnp.zeros((128, DHALF), jnp.float32)
    )
    h2 = _mm2(p1, degi, dego, b1.reshape(1, DH), W2, N)
    p2 = _agg_esplit_kernel(NP, K, EP, C2)(
        h2, src3w, dst3w, jnp.zeros((128, K), jnp.float32)
    )
    return _fin(p2, degi, b2.reshape(1, K), N)


# speedup vs baseline: 10.0295x; 1.0673x over previous
import functools

import jax
import jax.numpy as jnp
from jax import lax
from jax.experimental import pallas as pl
from jax.experimental.pallas import tpu as pltpu
from jax.experimental.pallas import tpu_sc as plsc

NS = 16
NCORES = 2


def _deg_kernel(NP, E, C):
    RPT = E // C // NS
    NPT = NP // NS
    mesh = plsc.VectorSubcoreMesh(core_axis_name="c", subcore_axis_name="s")

    @functools.partial(
        pl.kernel,
        out_type=[
            jax.ShapeDtypeStruct((NP,), jnp.float32),
            jax.ShapeDtypeStruct((NP,), jnp.float32),
        ],
        mesh=mesh,
        compiler_params=pltpu.CompilerParams(use_tc_tiling_on_sc=False),
        scratch_types=[
            pltpu.VMEM((RPT, C), jnp.int32),
            pltpu.VMEM((C,), jnp.float32),
            pltpu.VMEM((NPT,), jnp.float32),
            pltpu.VMEM_SHARED((NP,), jnp.float32),
        ],
    )
    def deg(ei4, ones_h, zeros_h, out0_h, out1_h, idx_v, ones_v, zer_v, acc_s):
        c = lax.axis_index("c")
        s = lax.axis_index("s")
        pltpu.sync_copy(zeros_h, zer_v)
        pltpu.sync_copy(zer_v, acc_s.at[pl.ds(s * NPT, NPT)])
        pltpu.sync_copy(ones_h, ones_v)
        pltpu.sync_copy(ei4.at[c, s], idx_v)
        plsc.subcore_barrier()

        @pl.loop(0, RPT)
        def _(j):
            pltpu.sync_copy(ones_v, acc_s.at[idx_v.at[j]], add=True)

        plsc.subcore_barrier()
        sl = pl.ds(s * NPT, NPT)
        pltpu.sync_copy(acc_s.at[sl], zer_v)

        @pl.when(c == 0)
        def _():
            pltpu.sync_copy(zer_v, out0_h.at[sl])

        @pl.when(c == 1)
        def _():
            pltpu.sync_copy(zer_v, out1_h.at[sl])

    return deg


def _edge_pipeline(h_h, src_v, dst_v, rows_a, rows_b, sem_a, sem_b, acc_s, RPT):
    pairs = RPT - (RPT % 2)
    pltpu.async_copy(h_h.at[src_v.at[0]], rows_a, sem_a)

    @pl.loop(0, pairs, step=2)
    def _(j):
        pltpu.async_copy(h_h.at[src_v.at[j + 1]], rows_b, sem_b)
        pltpu.make_async_copy(h_h.at[src_v.at[j]], rows_a, sem_a).wait()
        pltpu.sync_copy(rows_a, acc_s.at[dst_v.at[j]], add=True)

        @pl.when(j + 2 < RPT)
        def _():
            pltpu.async_copy(h_h.at[src_v.at[j + 2]], rows_a, sem_a)

        pltpu.make_async_copy(h_h.at[src_v.at[j + 1]], rows_b, sem_b).wait()
        pltpu.sync_copy(rows_b, acc_s.at[dst_v.at[j + 1]], add=True)

    if RPT % 2:
        pltpu.make_async_copy(h_h.at[src_v.at[RPT - 1]], rows_a, sem_a).wait()
        pltpu.sync_copy(rows_a, acc_s.at[dst_v.at[RPT - 1]], add=True)


def _agg_dsplit_kernel(NP, DHALF, EP, C):
    RPT = EP // C // NS
    NPT = NP // NS
    ZCH = 128
    assert NPT % ZCH == 0
    mesh = plsc.VectorSubcoreMesh(core_axis_name="c", subcore_axis_name="s")

    @functools.partial(
        pl.kernel,
        out_type=jax.ShapeDtypeStruct((NCORES, NP, DHALF), jnp.float32),
        mesh=mesh,
        compiler_params=pltpu.CompilerParams(use_tc_tiling_on_sc=False),
        scratch_types=[
            pltpu.VMEM((RPT, C), jnp.int32),
            pltpu.VMEM((RPT, C), jnp.int32),
            pltpu.VMEM((C, DHALF), jnp.float32),
            pltpu.VMEM((C, DHALF), jnp.float32),
            pltpu.VMEM((ZCH, DHALF), jnp.float32),
            pltpu.VMEM_SHARED((NP, DHALF), jnp.float32),
            pltpu.SemaphoreType.DMA,
            pltpu.SemaphoreType.DMA,
        ],
    )
    def agg(ha_h, hb_h, src3, dst3, zeros_h, out_h,
            src_v, dst_v, rows_a, rows_b, buf_v, acc_s, sem_a, sem_b):
        c = lax.axis_index("c")
        s = lax.axis_index("s")
        pltpu.sync_copy(zeros_h, buf_v)

        @pl.loop(0, NPT // ZCH)
        def _(k):
            pltpu.sync_copy(buf_v, acc_s.at[pl.ds(s * NPT + k * ZCH, ZCH)])

        pltpu.sync_copy(src3.at[s], src_v)
        pltpu.sync_copy(dst3.at[s], dst_v)
        plsc.subcore_barrier()

        @pl.when(c == 0)
        def _():
            _edge_pipeline(ha_h, src_v, dst_v, rows_a, rows_b, sem_a, sem_b,
                           acc_s, RPT)

        @pl.when(c == 1)
        def _():
            _edge_pipeline(hb_h, src_v, dst_v, rows_a, rows_b, sem_a, sem_b,
                           acc_s, RPT)

        plsc.subcore_barrier()

        @pl.loop(0, NPT // ZCH)
        def _(k):
            pltpu.sync_copy(acc_s.at[pl.ds(s * NPT + k * ZCH, ZCH)], buf_v)
            pltpu.sync_copy(buf_v, out_h.at[c, pl.ds(s * NPT + k * ZCH, ZCH)])

    return agg


def _agg_esplit_kernel(NP, D, EP, C):
    RPT = EP // NCORES // C // NS
    NPT = NP // NS
    ZCH = min(128, 8192 // D)
    assert NPT % ZCH == 0
    mesh = plsc.VectorSubcoreMesh(core_axis_name="c", subcore_axis_name="s")

    @functools.partial(
        pl.kernel,
        out_type=jax.ShapeDtypeStruct((NCORES, NP, D), jnp.float32),
        mesh=mesh,
        compiler_params=pltpu.CompilerParams(use_tc_tiling_on_sc=False),
        scratch_types=[
            pltpu.VMEM((RPT, C), jnp.int32),
            pltpu.VMEM((RPT, C), jnp.int32),
            pltpu.VMEM((C, D), jnp.float32),
            pltpu.VMEM((C, D), jnp.float32),
            pltpu.VMEM((ZCH, D), jnp.float32),
            pltpu.VMEM_SHARED((NP, D), jnp.float32),
            pltpu.SemaphoreType.DMA,
            pltpu.SemaphoreType.DMA,
        ],
    )
    def agg(h_h, src3, dst3, zeros_h, out_h,
            src_v, dst_v, rows_a, rows_b, buf_v, acc_s, sem_a, sem_b):
        c = lax.axis_index("c")
        s = lax.axis_index("s")
        w = c * NS + s
        pltpu.sync_copy(zeros_h, buf_v)

        @pl.loop(0, NPT // ZCH)
        def _(k):
            pltpu.sync_copy(buf_v, acc_s.at[pl.ds(s * NPT + k * ZCH, ZCH)])

        pltpu.sync_copy(src3.at[w], src_v)
        pltpu.sync_copy(dst3.at[w], dst_v)
        plsc.subcore_barrier()

        _edge_pipeline(h_h, src_v, dst_v, rows_a, rows_b, sem_a, sem_b, acc_s, RPT)

        plsc.subcore_barrier()

        @pl.loop(0, NPT // ZCH)
        def _(k):
            pltpu.sync_copy(acc_s.at[pl.ds(s * NPT + k * ZCH, ZCH)], buf_v)
            pltpu.sync_copy(buf_v, out_h.at[c, pl.ds(s * NPT + k * ZCH, ZCH)])

    return agg


def _mm1(x, dego, W1h, B=512):
    N, DI = x.shape
    DH = W1h.shape[1]

    def body(x_ref, d_ref, w_ref, o_ref):
        nrm = lax.rsqrt(jnp.maximum(d_ref[...], 1.0))
        o_ref[...] = jnp.dot(
            x_ref[...] * nrm, w_ref[...], preferred_element_type=jnp.float32
        )

    return pl.pallas_call(
        body,
        grid=(pl.cdiv(N, B),),
        in_specs=[
            pl.BlockSpec((B, DI), lambda i: (i, 0)),
            pl.BlockSpec((B, 1), lambda i: (i, 0)),
            pl.BlockSpec((DI, DH), lambda i: (0, 0)),
        ],
        out_specs=pl.BlockSpec((B, DH), lambda i: (i, 0)),
        out_shape=jax.ShapeDtypeStruct((N, DH), jnp.float32),
    )(x, dego, W1h)


def _mm2(p, degi, dego, b1, W2, N, B=512):
    _, NP, DH = p.shape
    K = W2.shape[1]

    def body(p_ref, di_ref, do_ref, b_ref, w_ref, o_ref):
        agg = p_ref[0] + p_ref[1]
        ndst = lax.rsqrt(jnp.maximum(di_ref[...], 1.0))
        h = jnp.maximum(agg * ndst + b_ref[...], 0.0)
        nsrc = lax.rsqrt(jnp.maximum(do_ref[...], 1.0))
        o_ref[...] = jnp.dot(h * nsrc, w_ref[...], preferred_element_type=jnp.float32)

    return pl.pallas_call(
        body,
        grid=(pl.cdiv(N, B),),
        in_specs=[
            pl.BlockSpec((2, B, DH), lambda i: (0, i, 0)),
            pl.BlockSpec((B, 1), lambda i: (i, 0)),
            pl.BlockSpec((B, 1), lambda i: (i, 0)),
            pl.BlockSpec((1, DH), lambda i: (0, 0)),
            pl.BlockSpec((DH, K), lambda i: (0, 0)),
        ],
        out_specs=pl.BlockSpec((B, K), lambda i: (i, 0)),
        out_shape=jax.ShapeDtypeStruct((N, K), jnp.float32),
    )(p, degi, dego, b1, W2)


def _fin(p, degi, b2, N, B=512):
    _, NP, K = p.shape

    def body(p_ref, di_ref, b_ref, o_ref):
        agg = p_ref[0] + p_ref[1]
        ndst = lax.rsqrt(jnp.maximum(di_ref[...], 1.0))
        o_ref[...] = agg * ndst + b_ref[...]

    return pl.pallas_call(
        body,
        grid=(pl.cdiv(N, B),),
        in_specs=[
            pl.BlockSpec((2, B, K), lambda i: (0, i, 0)),
            pl.BlockSpec((B, 1), lambda i: (i, 0)),
            pl.BlockSpec((1, K), lambda i: (0, 0)),
        ],
        out_specs=pl.BlockSpec((B, K), lambda i: (i, 0)),
        out_shape=jax.ShapeDtypeStruct((N, K), jnp.float32),
    )(p, degi, b2)


def kernel(features, edge_index, W1, b1, W2, b2):
    N, DI = features.shape
    E = edge_index.shape[1]
    DH = W1.shape[1]
    K = W2.shape[1]
    NP = ((N + 2047) // 2048) * 2048
    CD = 80
    C1 = 40
    C2 = 80
    assert E % (NS * CD) == 0 and NP % (NS * 8) == 0
    assert DH % 2 == 0 and K % 2 == 0

    GR = NCORES * NS * C2
    EP = ((E + GR - 1) // GR) * GR
    srcp = jnp.concatenate([edge_index[0], jnp.zeros((EP - E,), jnp.int32)])
    dstp = jnp.concatenate([edge_index[1], jnp.full((EP - E,), NP - 1, jnp.int32)])

    ei4 = edge_index.reshape(2, NS, E // NS // CD, CD)
    src3w1 = srcp.reshape(NCORES * NS, EP // NCORES // NS // C1, C1)
    dst3w1 = dstp.reshape(NCORES * NS, EP // NCORES // NS // C1, C1)
    src3w = srcp.reshape(NCORES * NS, EP // NCORES // NS // C2, C2)
    dst3w = dstp.reshape(NCORES * NS, EP // NCORES // NS // C2, C2)

    dego, degi = _deg_kernel(NP, E, CD)(
        ei4, jnp.ones((CD,), jnp.float32), jnp.zeros((NP // NS,), jnp.float32)
    )
    dego = dego.reshape(NP, 1)
    degi = degi.reshape(NP, 1)

    h1 = _mm1(features, dego, W1)
    p1 = _agg_esplit_kernel(NP, DH, EP, C1)(
        h1, src3w1, dst3w1, jnp.zeros((min(128, 8192 // DH), DH), jnp.float32)
    )
    h2 = _mm2(p1, degi, dego, b1.reshape(1, DH), W2, N)
    p2 = _agg_esplit_kernel(NP, K, EP, C2)(
        h2, src3w, dst3w, jnp.zeros((min(128, 8192 // K), K), jnp.float32)
    )
    return _fin(p2, degi, b2.reshape(1, K), N)


# speedup vs baseline: 11.2718x; 1.1239x over previous
import functools

import jax
import jax.numpy as jnp
from jax import lax
from jax.experimental import pallas as pl
from jax.experimental.pallas import tpu as pltpu
from jax.experimental.pallas import tpu_sc as plsc

NS = 16
NCORES = 2


def _deg_kernel(NP, E, C):
    RPT = E // C // NS
    NPT = NP // NS
    mesh = plsc.VectorSubcoreMesh(core_axis_name="c", subcore_axis_name="s")

    @functools.partial(
        pl.kernel,
        out_type=[
            jax.ShapeDtypeStruct((NP,), jnp.float32),
            jax.ShapeDtypeStruct((NP,), jnp.float32),
        ],
        mesh=mesh,
        compiler_params=pltpu.CompilerParams(use_tc_tiling_on_sc=False),
        scratch_types=[
            pltpu.VMEM((RPT, C), jnp.int32),
            pltpu.VMEM((C,), jnp.float32),
            pltpu.VMEM((NPT,), jnp.float32),
            pltpu.VMEM_SHARED((NP,), jnp.float32),
        ],
    )
    def deg(ei4, ones_h, zeros_h, out0_h, out1_h, idx_v, ones_v, zer_v, acc_s):
        c = lax.axis_index("c")
        s = lax.axis_index("s")
        pltpu.sync_copy(zeros_h, zer_v)
        pltpu.sync_copy(zer_v, acc_s.at[pl.ds(s * NPT, NPT)])
        pltpu.sync_copy(ones_h, ones_v)
        pltpu.sync_copy(ei4.at[c, s], idx_v)
        plsc.subcore_barrier()

        @pl.loop(0, RPT)
        def _(j):
            pltpu.sync_copy(ones_v, acc_s.at[idx_v.at[j]], add=True)

        plsc.subcore_barrier()
        sl = pl.ds(s * NPT, NPT)
        pltpu.sync_copy(acc_s.at[sl], zer_v)

        @pl.when(c == 0)
        def _():
            pltpu.sync_copy(zer_v, out0_h.at[sl])

        @pl.when(c == 1)
        def _():
            pltpu.sync_copy(zer_v, out1_h.at[sl])

    return deg


def _edge_pipeline(h_h, src_v, dst_v, rows_a, rows_b, sem_a, sem_b, acc_s, RPT):
    pairs = RPT - (RPT % 2)
    pltpu.async_copy(h_h.at[src_v.at[0]], rows_a, sem_a)

    @pl.loop(0, pairs, step=2)
    def _(j):
        pltpu.async_copy(h_h.at[src_v.at[j + 1]], rows_b, sem_b)
        pltpu.make_async_copy(h_h.at[src_v.at[j]], rows_a, sem_a).wait()
        pltpu.sync_copy(rows_a, acc_s.at[dst_v.at[j]], add=True)

        @pl.when(j + 2 < RPT)
        def _():
            pltpu.async_copy(h_h.at[src_v.at[j + 2]], rows_a, sem_a)

        pltpu.make_async_copy(h_h.at[src_v.at[j + 1]], rows_b, sem_b).wait()
        pltpu.sync_copy(rows_b, acc_s.at[dst_v.at[j + 1]], add=True)

    if RPT % 2:
        pltpu.make_async_copy(h_h.at[src_v.at[RPT - 1]], rows_a, sem_a).wait()
        pltpu.sync_copy(rows_a, acc_s.at[dst_v.at[RPT - 1]], add=True)


def _agg_dsplit_kernel(NP, DHALF, EP, C):
    RPT = EP // C // NS
    NPT = NP // NS
    ZCH = 128
    assert NPT % ZCH == 0
    mesh = plsc.VectorSubcoreMesh(core_axis_name="c", subcore_axis_name="s")

    @functools.partial(
        pl.kernel,
        out_type=jax.ShapeDtypeStruct((NCORES, NP, DHALF), jnp.float32),
        mesh=mesh,
        compiler_params=pltpu.CompilerParams(use_tc_tiling_on_sc=False),
        scratch_types=[
            pltpu.VMEM((RPT, C), jnp.int32),
            pltpu.VMEM((RPT, C), jnp.int32),
            pltpu.VMEM((C, DHALF), jnp.float32),
            pltpu.VMEM((C, DHALF), jnp.float32),
            pltpu.VMEM((ZCH, DHALF), jnp.float32),
            pltpu.VMEM_SHARED((NP, DHALF), jnp.float32),
            pltpu.SemaphoreType.DMA,
            pltpu.SemaphoreType.DMA,
        ],
    )
    def agg(ha_h, hb_h, src3, dst3, zeros_h, out_h,
            src_v, dst_v, rows_a, rows_b, buf_v, acc_s, sem_a, sem_b):
        c = lax.axis_index("c")
        s = lax.axis_index("s")
        pltpu.sync_copy(zeros_h, buf_v)

        @pl.loop(0, NPT // ZCH)
        def _(k):
            pltpu.sync_copy(buf_v, acc_s.at[pl.ds(s * NPT + k * ZCH, ZCH)])

        pltpu.sync_copy(src3.at[s], src_v)
        pltpu.sync_copy(dst3.at[s], dst_v)
        plsc.subcore_barrier()

        @pl.when(c == 0)
        def _():
            _edge_pipeline(ha_h, src_v, dst_v, rows_a, rows_b, sem_a, sem_b,
                           acc_s, RPT)

        @pl.when(c == 1)
        def _():
            _edge_pipeline(hb_h, src_v, dst_v, rows_a, rows_b, sem_a, sem_b,
                           acc_s, RPT)

        plsc.subcore_barrier()

        @pl.loop(0, NPT // ZCH)
        def _(k):
            pltpu.sync_copy(acc_s.at[pl.ds(s * NPT + k * ZCH, ZCH)], buf_v)
            pltpu.sync_copy(buf_v, out_h.at[c, pl.ds(s * NPT + k * ZCH, ZCH)])

    return agg


def _agg_esplit_kernel(NP, D, EP, C):
    RPT = EP // NCORES // C // NS
    NPT = NP // NS
    ZCH = min(128, 8192 // D)
    assert NPT % ZCH == 0
    mesh = plsc.VectorSubcoreMesh(core_axis_name="c", subcore_axis_name="s")

    @functools.partial(
        pl.kernel,
        out_type=jax.ShapeDtypeStruct((NCORES, NP, D), jnp.float32),
        mesh=mesh,
        compiler_params=pltpu.CompilerParams(use_tc_tiling_on_sc=False),
        scratch_types=[
            pltpu.VMEM((RPT, C), jnp.int32),
            pltpu.VMEM((RPT, C), jnp.int32),
            pltpu.VMEM((C, D), jnp.float32),
            pltpu.VMEM((C, D), jnp.float32),
            pltpu.VMEM((ZCH, D), jnp.float32),
            pltpu.VMEM_SHARED((NP, D), jnp.float32),
            pltpu.SemaphoreType.DMA,
            pltpu.SemaphoreType.DMA,
        ],
    )
    def agg(h_h, src3, dst3, zeros_h, out_h,
            src_v, dst_v, rows_a, rows_b, buf_v, acc_s, sem_a, sem_b):
        c = lax.axis_index("c")
        s = lax.axis_index("s")
        w = c * NS + s
        pltpu.sync_copy(zeros_h, buf_v)

        @pl.loop(0, NPT // ZCH)
        def _(k):
            pltpu.sync_copy(buf_v, acc_s.at[pl.ds(s * NPT + k * ZCH, ZCH)])

        pltpu.sync_copy(src3.at[w], src_v)
        pltpu.sync_copy(dst3.at[w], dst_v)
        plsc.subcore_barrier()

        _edge_pipeline(h_h, src_v, dst_v, rows_a, rows_b, sem_a, sem_b, acc_s, RPT)

        plsc.subcore_barrier()

        @pl.loop(0, NPT // ZCH)
        def _(k):
            pltpu.sync_copy(acc_s.at[pl.ds(s * NPT + k * ZCH, ZCH)], buf_v)
            pltpu.sync_copy(buf_v, out_h.at[c, pl.ds(s * NPT + k * ZCH, ZCH)])

    return agg


def _mm1(x, dego, W1h, B=512):
    N, DI = x.shape
    DH = W1h.shape[1]

    def body(x_ref, d_ref, w_ref, o_ref):
        nrm = lax.rsqrt(jnp.maximum(d_ref[...], 1.0))
        o_ref[...] = jnp.dot(
            x_ref[...] * nrm, w_ref[...], preferred_element_type=jnp.float32
        )

    return pl.pallas_call(
        body,
        grid=(pl.cdiv(N, B),),
        in_specs=[
            pl.BlockSpec((B, DI), lambda i: (i, 0)),
            pl.BlockSpec((B, 1), lambda i: (i, 0)),
            pl.BlockSpec((DI, DH), lambda i: (0, 0)),
        ],
        out_specs=pl.BlockSpec((B, DH), lambda i: (i, 0)),
        out_shape=jax.ShapeDtypeStruct((N, DH), jnp.float32),
    )(x, dego, W1h)


def _mm2(p, degi, dego, b1, W2, N, B=512):
    _, NP, DH = p.shape
    K = W2.shape[1]

    def body(p_ref, di_ref, do_ref, b_ref, w_ref, o_ref):
        agg = p_ref[0] + p_ref[1]
        ndst = lax.rsqrt(jnp.maximum(di_ref[...], 1.0))
        h = jnp.maximum(agg * ndst + b_ref[...], 0.0)
        nsrc = lax.rsqrt(jnp.maximum(do_ref[...], 1.0))
        o_ref[...] = jnp.dot(h * nsrc, w_ref[...], preferred_element_type=jnp.float32)

    return pl.pallas_call(
        body,
        grid=(pl.cdiv(N, B),),
        in_specs=[
            pl.BlockSpec((2, B, DH), lambda i: (0, i, 0)),
            pl.BlockSpec((B, 1), lambda i: (i, 0)),
            pl.BlockSpec((B, 1), lambda i: (i, 0)),
            pl.BlockSpec((1, DH), lambda i: (0, 0)),
            pl.BlockSpec((DH, K), lambda i: (0, 0)),
        ],
        out_specs=pl.BlockSpec((B, K), lambda i: (i, 0)),
        out_shape=jax.ShapeDtypeStruct((N, K), jnp.float32),
    )(p, degi, dego, b1, W2)


def _fin(p, degi, b2, N, B=512):
    _, NP, K = p.shape

    def body(p_ref, di_ref, b_ref, o_ref):
        agg = p_ref[0] + p_ref[1]
        ndst = lax.rsqrt(jnp.maximum(di_ref[...], 1.0))
        o_ref[...] = agg * ndst + b_ref[...]

    return pl.pallas_call(
        body,
        grid=(pl.cdiv(N, B),),
        in_specs=[
            pl.BlockSpec((2, B, K), lambda i: (0, i, 0)),
            pl.BlockSpec((B, 1), lambda i: (i, 0)),
            pl.BlockSpec((1, K), lambda i: (0, 0)),
        ],
        out_specs=pl.BlockSpec((B, K), lambda i: (i, 0)),
        out_shape=jax.ShapeDtypeStruct((N, K), jnp.float32),
    )(p, degi, b2)


def kernel(features, edge_index, W1, b1, W2, b2):
    N, DI = features.shape
    E = edge_index.shape[1]
    DH = W1.shape[1]
    K = W2.shape[1]
    NP = ((N + 2047) // 2048) * 2048
    CD = 80
    C1 = 80
    C2 = 80
    assert E % (NS * CD) == 0 and NP % (NS * 8) == 0
    assert DH % 2 == 0 and K % 2 == 0

    GR = NCORES * NS * C2
    EP = ((E + GR - 1) // GR) * GR
    srcp = jnp.concatenate([edge_index[0], jnp.zeros((EP - E,), jnp.int32)])
    dstp = jnp.concatenate([edge_index[1], jnp.full((EP - E,), NP - 1, jnp.int32)])

    ei4 = edge_index.reshape(2, NS, E // NS // CD, CD)
    src3w1 = srcp.reshape(NCORES * NS, EP // NCORES // NS // C1, C1)
    dst3w1 = dstp.reshape(NCORES * NS, EP // NCORES // NS // C1, C1)
    src3w = srcp.reshape(NCORES * NS, EP // NCORES // NS // C2, C2)
    dst3w = dstp.reshape(NCORES * NS, EP // NCORES // NS // C2, C2)

    dego, degi = _deg_kernel(NP, E, CD)(
        ei4, jnp.ones((CD,), jnp.float32), jnp.zeros((NP // NS,), jnp.float32)
    )
    dego = dego.reshape(NP, 1)
    degi = degi.reshape(NP, 1)

    h1 = _mm1(features, dego, W1)
    p1 = _agg_esplit_kernel(NP, DH, EP, C1)(
        h1, src3w1, dst3w1, jnp.zeros((min(128, 8192 // DH), DH), jnp.float32)
    )
    h2 = _mm2(p1, degi, dego, b1.reshape(1, DH), W2, N)
    p2 = _agg_esplit_kernel(NP, K, EP, C2)(
        h2, src3w, dst3w, jnp.zeros((min(128, 8192 // K), K), jnp.float32)
    )
    return _fin(p2, degi, b2.reshape(1, K), N)
